# baseline (speedup 1.0000x reference)
import functools

import jax
import jax.numpy as jnp
from jax import lax
from jax.experimental import pallas as pl
from jax.experimental.pallas import tpu as pltpu
from jax.experimental.pallas import tpu_sc as plsc

_BM = 256
_HC = 2
_POS_CHUNK = 256


def _router_body(x_ref, rw_ref, rb_ref, mask_ref, pos_ref, gates_ref, cnt_ref,
                 r1_ref, r2_ref):
    S, E = mask_ref.shape
    logits = jnp.dot(x_ref[...], rw_ref[...]) + rb_ref[...]

    iota_e = lax.broadcasted_iota(jnp.int32, (S, E), 1)
    big = jnp.int32(E)
    v1 = jnp.max(logits, axis=1, keepdims=True)
    idx1 = jnp.min(jnp.where(logits == v1, iota_e, big), axis=1, keepdims=True)
    oh1 = iota_e == idx1
    l2 = jnp.where(oh1, jnp.float32(-1e30), logits)
    v2 = jnp.max(l2, axis=1, keepdims=True)
    idx2 = jnp.min(jnp.where(l2 == v2, iota_e, big), axis=1, keepdims=True)
    oh2 = iota_e == idx2
    maskb = oh1 | oh2
    mask = maskb.astype(jnp.float32)

    denom = 1.0 + jnp.exp(v2 - v1)
    gates = jnp.where(maskb, jnp.exp(logits - v1) / denom, 0.0)

    mask_ref[...] = mask
    gates_ref[...] = gates
    cnt_ref[...] = jnp.sum(mask, axis=0, keepdims=True).astype(jnp.int32)

    for c in range(S // _POS_CHUNK):
        row_t = lax.broadcasted_iota(jnp.int32, (_POS_CHUNK, S), 0) + c * _POS_CHUNK
        col_t = lax.broadcasted_iota(jnp.int32, (_POS_CHUNK, S), 1)
        lt = (col_t < row_t).astype(jnp.float32)
        pos_ref[c * _POS_CHUNK:(c + 1) * _POS_CHUNK, :] = jnp.dot(
            lt, mask, preferred_element_type=jnp.float32)

    cntf = jnp.sum(mask, axis=0, keepdims=True)
    nb = jnp.floor((cntf + (_BM - 1.0)) * (1.0 / _BM))
    iu = lax.broadcasted_iota(jnp.int32, (E, E), 0)
    iv = lax.broadcasted_iota(jnp.int32, (E, E), 1)
    ut = (iu < iv).astype(jnp.float32)
    bstart = jnp.dot(nb, ut, preferred_element_type=jnp.float32)
    rfull = bstart * jnp.float32(_BM) + pos_ref[...]
    r1_ref[...] = jnp.sum(jnp.where(oh1, rfull, 0.0), axis=1,
                          keepdims=True).astype(jnp.int32)
    r2_ref[...] = jnp.sum(jnp.where(oh2, rfull, 0.0), axis=1,
                          keepdims=True).astype(jnp.int32)


def _make_sc_scatter(S, D, GBM):
    info = plsc.get_sparse_core_info()
    nw = info.num_cores * info.num_subcores
    tpw = S // nw
    mesh = plsc.VectorSubcoreMesh(core_axis_name="c", subcore_axis_name="s")

    @functools.partial(
        pl.kernel, mesh=mesh,
        out_type=jax.ShapeDtypeStruct((GBM, D), jnp.float32),
        scratch_types=[
            pltpu.VMEM((tpw,), jnp.int32),
            pltpu.VMEM((tpw,), jnp.int32),
            pltpu.VMEM((tpw, D), jnp.float32),
            pltpu.SemaphoreType.DMA,
        ],
    )
    def sc_scatter(x_hbm, r1_hbm, r2_hbm, xs_hbm, i1_v, i2_v, rows_v, sem):
        wid = lax.axis_index("s") * info.num_cores + lax.axis_index("c")
        base = wid * tpw
        pltpu.sync_copy(r1_hbm.at[pl.ds(base, tpw)], i1_v)
        pltpu.sync_copy(r2_hbm.at[pl.ds(base, tpw)], i2_v)
        pltpu.sync_copy(x_hbm.at[pl.ds(base, tpw)], rows_v)
        pltpu.async_copy(rows_v, xs_hbm.at[i1_v], sem).wait()
        pltpu.async_copy(rows_v, xs_hbm.at[i2_v], sem).wait()

    return sc_scatter


def _make_sc_combine(S, D, BM):
    info = plsc.get_sparse_core_info()
    nw = info.num_cores * info.num_subcores
    tpw = S // nw
    ch = 16
    mesh = plsc.VectorSubcoreMesh(core_axis_name="c", subcore_axis_name="s")

    @functools.partial(
        pl.kernel, mesh=mesh,
        out_type=jax.ShapeDtypeStruct((S, D), jnp.float32),
        scratch_types=[
            pltpu.VMEM((tpw,), jnp.int32),
            pltpu.VMEM((tpw,), jnp.int32),
            pltpu.VMEM((ch, D), jnp.float32),
            pltpu.VMEM((ch, D), jnp.float32),
            pltpu.VMEM((ch, D), jnp.float32),
            pltpu.VMEM((ch, D), jnp.float32),
            pltpu.VMEM((ch, D), jnp.float32),
            pltpu.SemaphoreType.DMA,
        ],
    )
    def sc_combine(ys_hbm, r1_hbm, r2_hbm, out_hbm,
                   i1_v, i2_v, a_v, b_v, c_v, d_v, o_v, sem):
        wid = lax.axis_index("s") * info.num_cores + lax.axis_index("c")
        base = wid * tpw
        pltpu.sync_copy(r1_hbm.at[pl.ds(base, tpw)], i1_v)
        pltpu.sync_copy(r2_hbm.at[pl.ds(base, tpw)], i2_v)
        for k in range(tpw // ch):
            r1v = i1_v[pl.ds(k * ch, ch)]
            r2v = i2_v[pl.ds(k * ch, ch)]
            shift = BM.bit_length() - 1
            row1 = r1v + lax.shift_right_logical(r1v, shift) * BM
            row2 = r2v + lax.shift_right_logical(r2v, shift) * BM
            pltpu.async_copy(ys_hbm.at[row1], a_v, sem).wait()
            pltpu.async_copy(ys_hbm.at[row1 + BM], b_v, sem).wait()
            pltpu.async_copy(ys_hbm.at[row2], c_v, sem).wait()
            pltpu.async_copy(ys_hbm.at[row2 + BM], d_v, sem).wait()
            for t in range(ch):
                def body(c2, carry):
                    sl = pl.ds(c2 * 16, 16)
                    o_v[t, sl] = (a_v[t, sl] + b_v[t, sl]
                                  + c_v[t, sl] + d_v[t, sl])
                    return carry
                lax.fori_loop(0, D // 16, body, 0)
            pltpu.sync_copy(o_v, out_hbm.at[pl.ds(base + k * ch, ch)])

    return sc_combine


def _ffn_body(be_ref, bj_ref, nblk_ref,
              xs_ref, mask_ref, pos_ref, gates_ref,
              fc1w_ref, fc1b_ref, fc2w_ref, fc2b_ref, out_ref,
              w1s_ref, w2s_ref):
    hc = pl.program_id(0)
    g = pl.program_id(1)
    S, E = mask_ref.shape

    e = be_ref[g]
    j = bj_ref[g]

    @pl.when(g < nblk_ref[0])
    def _block():
        @pl.when(j == 0)
        def _cast():
            w1s_ref[...] = fc1w_ref[0].astype(jnp.bfloat16)
            w2s_ref[...] = fc2w_ref[0].astype(jnp.bfloat16)

        lane = lax.broadcasted_iota(jnp.int32, (S, E), 1)
        is_e = lane == e
        mcol = jnp.sum(jnp.where(is_e, mask_ref[...], 0.0), axis=1, keepdims=True)
        pcol = jnp.sum(jnp.where(is_e, pos_ref[...], 0.0), axis=1, keepdims=True)
        gcol = jnp.sum(jnp.where(is_e, gates_ref[...], 0.0), axis=1, keepdims=True)

        rid = (lax.broadcasted_iota(jnp.int32, (S, _BM), 1)
               + j * _BM).astype(jnp.float32)
        selT = jnp.where((pcol == rid) & (mcol > 0), 1.0, 0.0)

        xg = xs_ref[...]

        h = jnp.dot(xg.astype(jnp.bfloat16), w1s_ref[...],
                    preferred_element_type=jnp.float32)
        h = h + fc1b_ref[0]
        h = 0.5 * h * (1.0 + lax.erf(h * jnp.float32(0.7071067811865476)))
        y = jnp.dot(h.astype(jnp.bfloat16), w2s_ref[...],
                    preferred_element_type=jnp.float32)
        y = y + fc2b_ref[0] * (hc == 0).astype(jnp.float32)

        gval = lax.dot_general(selT, gcol, (((0,), (0,)), ((), ())),
                               preferred_element_type=jnp.float32)
        valid = lax.dot_general(selT, mcol * 0.0 + 1.0,
                                (((0,), (0,)), ((), ())),
                                preferred_element_type=jnp.float32)
        out_ref[...] = jnp.where(valid > 0, y * gval, 0.0)


def kernel(inputs, router_w, router_b, fc1_w, fc1_b, fc2_w, fc2_b):
    B, S0, D = inputs.shape
    E = router_w.shape[1]
    H = fc1_w.shape[2]
    S = B * S0

    x = inputs.reshape(S, D)
    rb = router_b.reshape(1, E)

    mask, pos, gates, counts, r1, r2 = pl.pallas_call(
        _router_body,
        out_shape=(
            jax.ShapeDtypeStruct((S, E), jnp.float32),
            jax.ShapeDtypeStruct((S, E), jnp.float32),
            jax.ShapeDtypeStruct((S, E), jnp.float32),
            jax.ShapeDtypeStruct((1, E), jnp.int32),
            jax.ShapeDtypeStruct((S, 1), jnp.int32),
            jax.ShapeDtypeStruct((S, 1), jnp.int32),
        ),
    )(x, router_w, rb)

    G = (2 * S) // _BM + E
    c = counts.reshape(E)
    nb = (c + (_BM - 1)) // _BM
    nb_end = jnp.cumsum(nb)
    nblk = nb_end[-1]
    gids = jnp.arange(G, dtype=jnp.int32)
    be = jnp.searchsorted(nb_end, gids, side="right").astype(jnp.int32)
    bstart = nb_end - nb
    safe_e = jnp.minimum(be, E - 1)
    bj = gids - bstart[safe_e]
    last_e = jnp.clip(be[jnp.maximum(nblk - 1, 0)], 0, E - 1)
    be = jnp.where(gids < nblk, safe_e, last_e).astype(jnp.int32)
    bj = jnp.where(gids < nblk, bj, 0).astype(jnp.int32)
    nblk_arr = jnp.full((1,), nblk, dtype=jnp.int32)

    xs = _make_sc_scatter(S, D, G * _BM)(x, r1.reshape(S), r2.reshape(S))

    hchunk = H // _HC
    grid_spec = pltpu.PrefetchScalarGridSpec(
        num_scalar_prefetch=3,
        grid=(_HC, G),
        in_specs=[
            pl.BlockSpec((_BM, D), lambda hc, g, be, bj, nblk: (g, 0)),
            pl.BlockSpec((S, E), lambda hc, g, be, bj, nblk: (0, 0)),
            pl.BlockSpec((S, E), lambda hc, g, be, bj, nblk: (0, 0)),
            pl.BlockSpec((S, E), lambda hc, g, be, bj, nblk: (0, 0)),
            pl.BlockSpec((1, D, hchunk),
                         lambda hc, g, be, bj, nblk: (be[g], 0, hc)),
            pl.BlockSpec((1, 1, hchunk),
                         lambda hc, g, be, bj, nblk: (be[g], 0, hc)),
            pl.BlockSpec((1, hchunk, D),
                         lambda hc, g, be, bj, nblk: (be[g], hc, 0)),
            pl.BlockSpec((1, 1, D),
                         lambda hc, g, be, bj, nblk: (be[g], 0, 0)),
        ],
        out_specs=pl.BlockSpec((_BM, D),
                               lambda hc, g, be, bj, nblk: (2 * g + hc, 0)),
        scratch_shapes=[
            pltpu.VMEM((D, hchunk), jnp.bfloat16),
            pltpu.VMEM((hchunk, D), jnp.bfloat16),
        ],
    )
    ys = pl.pallas_call(
        _ffn_body,
        grid_spec=grid_spec,
        out_shape=jax.ShapeDtypeStruct((2 * G * _BM, D), jnp.float32),
        compiler_params=pltpu.CompilerParams(
            dimension_semantics=("arbitrary", "arbitrary")),
    )(
        be, bj, nblk_arr,
        xs,
        mask, pos, gates,
        fc1_w,
        fc1_b.reshape(E, 1, H),
        fc2_w,
        fc2_b.reshape(E, 1, D),
    )
    out = _make_sc_combine(S, D, _BM)(ys, r1.reshape(S), r2.reshape(S))
    return out.reshape(B, S0, D)


# speedup vs baseline: 1.0209x; 1.0209x over previous
import functools

import jax
import jax.numpy as jnp
from jax import lax
from jax.experimental import pallas as pl
from jax.experimental.pallas import tpu as pltpu
from jax.experimental.pallas import tpu_sc as plsc

_BM = 256
_HC = 2
_POS_CHUNK = 256


def _router_body(x_ref, rw_ref, rb_ref, mask_ref, pos_ref, gates_ref, cnt_ref,
                 r1_ref, r2_ref):
    S, E = mask_ref.shape
    logits = jnp.dot(x_ref[...], rw_ref[...]) + rb_ref[...]

    iota_e = lax.broadcasted_iota(jnp.int32, (S, E), 1)
    big = jnp.int32(E)
    v1 = jnp.max(logits, axis=1, keepdims=True)
    idx1 = jnp.min(jnp.where(logits == v1, iota_e, big), axis=1, keepdims=True)
    oh1 = iota_e == idx1
    l2 = jnp.where(oh1, jnp.float32(-1e30), logits)
    v2 = jnp.max(l2, axis=1, keepdims=True)
    idx2 = jnp.min(jnp.where(l2 == v2, iota_e, big), axis=1, keepdims=True)
    oh2 = iota_e == idx2
    maskb = oh1 | oh2
    mask = maskb.astype(jnp.float32)

    denom = 1.0 + jnp.exp(v2 - v1)
    gates = jnp.where(maskb, jnp.exp(logits - v1) / denom, 0.0)

    mask_ref[...] = mask
    gates_ref[...] = gates
    cnt_ref[...] = jnp.sum(mask, axis=0, keepdims=True).astype(jnp.int32)

    for c in range(S // _POS_CHUNK):
        row_t = lax.broadcasted_iota(jnp.int32, (_POS_CHUNK, S), 0) + c * _POS_CHUNK
        col_t = lax.broadcasted_iota(jnp.int32, (_POS_CHUNK, S), 1)
        lt = (col_t < row_t).astype(jnp.float32)
        pos_ref[c * _POS_CHUNK:(c + 1) * _POS_CHUNK, :] = jnp.dot(
            lt, mask, preferred_element_type=jnp.float32)

    cntf = jnp.sum(mask, axis=0, keepdims=True)
    nb = jnp.floor((cntf + (_BM - 1.0)) * (1.0 / _BM))
    iu = lax.broadcasted_iota(jnp.int32, (E, E), 0)
    iv = lax.broadcasted_iota(jnp.int32, (E, E), 1)
    ut = (iu < iv).astype(jnp.float32)
    bstart = jnp.dot(nb, ut, preferred_element_type=jnp.float32)
    rfull = bstart * jnp.float32(_BM) + pos_ref[...]
    r1_ref[...] = jnp.sum(jnp.where(oh1, rfull, 0.0), axis=1,
                          keepdims=True).astype(jnp.int32)
    r2_ref[...] = jnp.sum(jnp.where(oh2, rfull, 0.0), axis=1,
                          keepdims=True).astype(jnp.int32)


def _make_sc_scatter(S, D, GBM):
    info = plsc.get_sparse_core_info()
    nw = info.num_cores * info.num_subcores
    tpw = S // nw
    mesh = plsc.VectorSubcoreMesh(core_axis_name="c", subcore_axis_name="s")

    @functools.partial(
        pl.kernel, mesh=mesh,
        out_type=jax.ShapeDtypeStruct((GBM, D), jnp.float32),
        scratch_types=[
            pltpu.VMEM((tpw,), jnp.int32),
            pltpu.VMEM((tpw,), jnp.int32),
            pltpu.VMEM((tpw, D), jnp.float32),
            pltpu.SemaphoreType.DMA,
        ],
    )
    def sc_scatter(x_hbm, r1_hbm, r2_hbm, xs_hbm, i1_v, i2_v, rows_v, sem):
        wid = lax.axis_index("s") * info.num_cores + lax.axis_index("c")
        base = wid * tpw
        pltpu.sync_copy(r1_hbm.at[pl.ds(base, tpw)], i1_v)
        pltpu.sync_copy(r2_hbm.at[pl.ds(base, tpw)], i2_v)
        pltpu.sync_copy(x_hbm.at[pl.ds(base, tpw)], rows_v)
        pltpu.async_copy(rows_v, xs_hbm.at[i1_v], sem).wait()
        pltpu.async_copy(rows_v, xs_hbm.at[i2_v], sem).wait()

    return sc_scatter


def _make_sc_combine(S, D, BM):
    info = plsc.get_sparse_core_info()
    nw = info.num_cores * info.num_subcores
    tpw = S // nw
    ch = 16
    mesh = plsc.VectorSubcoreMesh(core_axis_name="c", subcore_axis_name="s")

    @functools.partial(
        pl.kernel, mesh=mesh,
        out_type=jax.ShapeDtypeStruct((S, D), jnp.float32),
        scratch_types=[
            pltpu.VMEM((tpw,), jnp.int32),
            pltpu.VMEM((tpw,), jnp.int32),
            pltpu.VMEM((4 * ch,), jnp.int32),
            pltpu.VMEM((4 * ch, D), jnp.float32),
            pltpu.VMEM((ch, D), jnp.float32),
            pltpu.SemaphoreType.DMA,
        ],
    )
    def sc_combine(ys_hbm, r1_hbm, r2_hbm, out_hbm,
                   i1_v, i2_v, idx_v, rows_v, o_v, sem):
        wid = lax.axis_index("s") * info.num_cores + lax.axis_index("c")
        base = wid * tpw
        pltpu.sync_copy(r1_hbm.at[pl.ds(base, tpw)], i1_v)
        pltpu.sync_copy(r2_hbm.at[pl.ds(base, tpw)], i2_v)
        shift = BM.bit_length() - 1
        for k in range(tpw // ch):
            r1v = i1_v[pl.ds(k * ch, ch)]
            r2v = i2_v[pl.ds(k * ch, ch)]
            row1 = r1v + lax.shift_right_logical(r1v, shift) * BM
            row2 = r2v + lax.shift_right_logical(r2v, shift) * BM
            idx_v[pl.ds(0, ch)] = row1
            idx_v[pl.ds(ch, ch)] = row1 + BM
            idx_v[pl.ds(2 * ch, ch)] = row2
            idx_v[pl.ds(3 * ch, ch)] = row2 + BM
            pltpu.async_copy(ys_hbm.at[idx_v], rows_v, sem).wait()
            for t in range(ch):
                def body(c2, carry):
                    sl = pl.ds(c2 * 16, 16)
                    o_v[t, sl] = (rows_v[t, sl] + rows_v[ch + t, sl]
                                  + rows_v[2 * ch + t, sl]
                                  + rows_v[3 * ch + t, sl])
                    return carry
                lax.fori_loop(0, D // 16, body, 0)
            pltpu.sync_copy(o_v, out_hbm.at[pl.ds(base + k * ch, ch)])

    return sc_combine


def _ffn_body(be_ref, bj_ref, nblk_ref,
              xs_ref, mask_ref, pos_ref, gates_ref,
              fc1w_ref, fc1b_ref, fc2w_ref, fc2b_ref, out_ref,
              w1s_ref, w2s_ref):
    hc = pl.program_id(0)
    g = pl.program_id(1)
    S, E = mask_ref.shape

    e = be_ref[g]
    j = bj_ref[g]

    @pl.when(g < nblk_ref[0])
    def _block():
        @pl.when(j == 0)
        def _cast():
            w1s_ref[...] = fc1w_ref[0].astype(jnp.bfloat16)
            w2s_ref[...] = fc2w_ref[0].astype(jnp.bfloat16)

        lane = lax.broadcasted_iota(jnp.int32, (S, E), 1)
        is_e = lane == e
        mcol = jnp.sum(jnp.where(is_e, mask_ref[...], 0.0), axis=1, keepdims=True)
        pcol = jnp.sum(jnp.where(is_e, pos_ref[...], 0.0), axis=1, keepdims=True)
        gcol = jnp.sum(jnp.where(is_e, gates_ref[...], 0.0), axis=1, keepdims=True)

        rid = (lax.broadcasted_iota(jnp.int32, (S, _BM), 1)
               + j * _BM).astype(jnp.float32)
        selT = jnp.where((pcol == rid) & (mcol > 0), 1.0, 0.0)

        xg = xs_ref[...]

        h = jnp.dot(xg.astype(jnp.bfloat16), w1s_ref[...],
                    preferred_element_type=jnp.float32)
        h = h + fc1b_ref[0]
        h = 0.5 * h * (1.0 + lax.erf(h * jnp.float32(0.7071067811865476)))
        y = jnp.dot(h.astype(jnp.bfloat16), w2s_ref[...],
                    preferred_element_type=jnp.float32)
        y = y + fc2b_ref[0] * (hc == 0).astype(jnp.float32)

        gval = lax.dot_general(selT, gcol, (((0,), (0,)), ((), ())),
                               preferred_element_type=jnp.float32)
        valid = lax.dot_general(selT, mcol * 0.0 + 1.0,
                                (((0,), (0,)), ((), ())),
                                preferred_element_type=jnp.float32)
        out_ref[...] = jnp.where(valid > 0, y * gval, 0.0)


def kernel(inputs, router_w, router_b, fc1_w, fc1_b, fc2_w, fc2_b):
    B, S0, D = inputs.shape
    E = router_w.shape[1]
    H = fc1_w.shape[2]
    S = B * S0

    x = inputs.reshape(S, D)
    rb = router_b.reshape(1, E)

    mask, pos, gates, counts, r1, r2 = pl.pallas_call(
        _router_body,
        out_shape=(
            jax.ShapeDtypeStruct((S, E), jnp.float32),
            jax.ShapeDtypeStruct((S, E), jnp.float32),
            jax.ShapeDtypeStruct((S, E), jnp.float32),
            jax.ShapeDtypeStruct((1, E), jnp.int32),
            jax.ShapeDtypeStruct((S, 1), jnp.int32),
            jax.ShapeDtypeStruct((S, 1), jnp.int32),
        ),
    )(x, router_w, rb)

    G = (2 * S) // _BM + E
    c = counts.reshape(E)
    nb = (c + (_BM - 1)) // _BM
    nb_end = jnp.cumsum(nb)
    nblk = nb_end[-1]
    gids = jnp.arange(G, dtype=jnp.int32)
    be = jnp.searchsorted(nb_end, gids, side="right").astype(jnp.int32)
    bstart = nb_end - nb
    safe_e = jnp.minimum(be, E - 1)
    bj = gids - bstart[safe_e]
    last_e = jnp.clip(be[jnp.maximum(nblk - 1, 0)], 0, E - 1)
    be = jnp.where(gids < nblk, safe_e, last_e).astype(jnp.int32)
    bj = jnp.where(gids < nblk, bj, 0).astype(jnp.int32)
    nblk_arr = jnp.full((1,), nblk, dtype=jnp.int32)

    xs = _make_sc_scatter(S, D, G * _BM)(x, r1.reshape(S), r2.reshape(S))

    hchunk = H // _HC
    grid_spec = pltpu.PrefetchScalarGridSpec(
        num_scalar_prefetch=3,
        grid=(_HC, G),
        in_specs=[
            pl.BlockSpec((_BM, D), lambda hc, g, be, bj, nblk: (g, 0)),
            pl.BlockSpec((S, E), lambda hc, g, be, bj, nblk: (0, 0)),
            pl.BlockSpec((S, E), lambda hc, g, be, bj, nblk: (0, 0)),
            pl.BlockSpec((S, E), lambda hc, g, be, bj, nblk: (0, 0)),
            pl.BlockSpec((1, D, hchunk),
                         lambda hc, g, be, bj, nblk: (be[g], 0, hc)),
            pl.BlockSpec((1, 1, hchunk),
                         lambda hc, g, be, bj, nblk: (be[g], 0, hc)),
            pl.BlockSpec((1, hchunk, D),
                         lambda hc, g, be, bj, nblk: (be[g], hc, 0)),
            pl.BlockSpec((1, 1, D),
                         lambda hc, g, be, bj, nblk: (be[g], 0, 0)),
        ],
        out_specs=pl.BlockSpec((_BM, D),
                               lambda hc, g, be, bj, nblk: (2 * g + hc, 0)),
        scratch_shapes=[
            pltpu.VMEM((D, hchunk), jnp.bfloat16),
            pltpu.VMEM((hchunk, D), jnp.bfloat16),
        ],
    )
    ys = pl.pallas_call(
        _ffn_body,
        grid_spec=grid_spec,
        out_shape=jax.ShapeDtypeStruct((2 * G * _BM, D), jnp.float32),
        compiler_params=pltpu.CompilerParams(
            dimension_semantics=("arbitrary", "arbitrary")),
    )(
        be, bj, nblk_arr,
        xs,
        mask, pos, gates,
        fc1_w,
        fc1_b.reshape(E, 1, H),
        fc2_w,
        fc2_b.reshape(E, 1, D),
    )
    out = _make_sc_combine(S, D, _BM)(ys, r1.reshape(S), r2.reshape(S))
    return out.reshape(B, S0, D)


# speedup vs baseline: 1.0930x; 1.0707x over previous
import functools

import jax
import jax.numpy as jnp
from jax import lax
from jax.experimental import pallas as pl
from jax.experimental.pallas import tpu as pltpu
from jax.experimental.pallas import tpu_sc as plsc

_BM = 256
_HC = 2
_POS_CHUNK = 256


def _router_body(x_ref, rw_ref, rb_ref, mask_ref, pos_ref, gates_ref, cnt_ref,
                 r1_ref, r2_ref):
    S, E = mask_ref.shape
    logits = jnp.dot(x_ref[...], rw_ref[...]) + rb_ref[...]

    iota_e = lax.broadcasted_iota(jnp.int32, (S, E), 1)
    big = jnp.int32(E)
    v1 = jnp.max(logits, axis=1, keepdims=True)
    idx1 = jnp.min(jnp.where(logits == v1, iota_e, big), axis=1, keepdims=True)
    oh1 = iota_e == idx1
    l2 = jnp.where(oh1, jnp.float32(-1e30), logits)
    v2 = jnp.max(l2, axis=1, keepdims=True)
    idx2 = jnp.min(jnp.where(l2 == v2, iota_e, big), axis=1, keepdims=True)
    oh2 = iota_e == idx2
    maskb = oh1 | oh2
    mask = maskb.astype(jnp.float32)

    denom = 1.0 + jnp.exp(v2 - v1)
    gates = jnp.where(maskb, jnp.exp(logits - v1) / denom, 0.0)

    mask_ref[...] = mask
    gates_ref[...] = gates
    cnt_ref[...] = jnp.sum(mask, axis=0, keepdims=True).astype(jnp.int32)

    for c in range(S // _POS_CHUNK):
        row_t = lax.broadcasted_iota(jnp.int32, (_POS_CHUNK, S), 0) + c * _POS_CHUNK
        col_t = lax.broadcasted_iota(jnp.int32, (_POS_CHUNK, S), 1)
        lt = (col_t < row_t).astype(jnp.float32)
        pos_ref[c * _POS_CHUNK:(c + 1) * _POS_CHUNK, :] = jnp.dot(
            lt, mask, preferred_element_type=jnp.float32)

    cntf = jnp.sum(mask, axis=0, keepdims=True)
    nb = jnp.floor((cntf + (_BM - 1.0)) * (1.0 / _BM))
    iu = lax.broadcasted_iota(jnp.int32, (E, E), 0)
    iv = lax.broadcasted_iota(jnp.int32, (E, E), 1)
    ut = (iu < iv).astype(jnp.float32)
    bstart = jnp.dot(nb, ut, preferred_element_type=jnp.float32)
    rfull = bstart * jnp.float32(_BM) + pos_ref[...]
    r1_ref[...] = jnp.sum(jnp.where(oh1, rfull, 0.0), axis=1,
                          keepdims=True).astype(jnp.int32)
    r2_ref[...] = jnp.sum(jnp.where(oh2, rfull, 0.0), axis=1,
                          keepdims=True).astype(jnp.int32)


def _make_sc_scatter(S, D, GBM):
    info = plsc.get_sparse_core_info()
    nw = info.num_cores * info.num_subcores
    tpw = S // nw
    mesh = plsc.VectorSubcoreMesh(core_axis_name="c", subcore_axis_name="s")

    @functools.partial(
        pl.kernel, mesh=mesh,
        out_type=jax.ShapeDtypeStruct((GBM, D), jnp.float32),
        scratch_types=[
            pltpu.VMEM((tpw,), jnp.int32),
            pltpu.VMEM((tpw,), jnp.int32),
            pltpu.VMEM((tpw, D), jnp.float32),
            pltpu.SemaphoreType.DMA,
        ],
    )
    def sc_scatter(x_hbm, r1_hbm, r2_hbm, xs_hbm, i1_v, i2_v, rows_v, sem):
        wid = lax.axis_index("s") * info.num_cores + lax.axis_index("c")
        base = wid * tpw
        pltpu.sync_copy(r1_hbm.at[pl.ds(base, tpw)], i1_v)
        pltpu.sync_copy(r2_hbm.at[pl.ds(base, tpw)], i2_v)
        pltpu.sync_copy(x_hbm.at[pl.ds(base, tpw)], rows_v)
        pltpu.async_copy(rows_v, xs_hbm.at[i1_v], sem).wait()
        pltpu.async_copy(rows_v, xs_hbm.at[i2_v], sem).wait()

    return sc_scatter


def _ffn_body(be_ref, bj_ref, nblk_ref,
              xs_ref, mask_ref, pos_ref, gates_ref,
              fc1w_ref, fc1b_ref, fc2w_ref, fc2b_ref, out_ref,
              w1s_ref, w2s_ref):
    hc = pl.program_id(0)
    g = pl.program_id(1)
    S, E = mask_ref.shape

    @pl.when((hc == 0) & (g == 0))
    def _init():
        out_ref[...] = jnp.zeros_like(out_ref)

    e = be_ref[g]
    j = bj_ref[g]

    @pl.when(g < nblk_ref[0])
    def _block():
        @pl.when(j == 0)
        def _cast():
            w1s_ref[...] = fc1w_ref[0].astype(jnp.bfloat16)
            w2s_ref[...] = fc2w_ref[0].astype(jnp.bfloat16)

        lane = lax.broadcasted_iota(jnp.int32, (S, E), 1)
        is_e = lane == e
        mcol = jnp.sum(jnp.where(is_e, mask_ref[...], 0.0), axis=1, keepdims=True)
        pcol = jnp.sum(jnp.where(is_e, pos_ref[...], 0.0), axis=1, keepdims=True)
        gcol = jnp.sum(jnp.where(is_e, gates_ref[...], 0.0), axis=1, keepdims=True)

        rid = (lax.broadcasted_iota(jnp.int32, (S, _BM), 1)
               + j * _BM).astype(jnp.float32)
        selT = jnp.where((pcol == rid) & (mcol > 0), 1.0, 0.0)

        xg = xs_ref[...]

        h = jnp.dot(xg.astype(jnp.bfloat16), w1s_ref[...],
                    preferred_element_type=jnp.float32)
        h = h + fc1b_ref[0]
        h = 0.5 * h * (1.0 + lax.erf(h * jnp.float32(0.7071067811865476)))
        y = jnp.dot(h.astype(jnp.bfloat16), w2s_ref[...],
                    preferred_element_type=jnp.float32)
        y = y + fc2b_ref[0] * (hc == 0).astype(jnp.float32)

        gval = lax.dot_general(selT, gcol, (((0,), (0,)), ((), ())),
                               preferred_element_type=jnp.float32)
        valid = lax.dot_general(selT, mcol * 0.0 + 1.0,
                                (((0,), (0,)), ((), ())),
                                preferred_element_type=jnp.float32)
        yg = jnp.where(valid > 0, y * gval, 0.0)

        out_ref[...] += jnp.dot(selT.astype(jnp.bfloat16),
                                yg.astype(jnp.bfloat16),
                                preferred_element_type=jnp.float32)


def kernel(inputs, router_w, router_b, fc1_w, fc1_b, fc2_w, fc2_b):
    B, S0, D = inputs.shape
    E = router_w.shape[1]
    H = fc1_w.shape[2]
    S = B * S0

    x = inputs.reshape(S, D)
    rb = router_b.reshape(1, E)

    mask, pos, gates, counts, r1, r2 = pl.pallas_call(
        _router_body,
        out_shape=(
            jax.ShapeDtypeStruct((S, E), jnp.float32),
            jax.ShapeDtypeStruct((S, E), jnp.float32),
            jax.ShapeDtypeStruct((S, E), jnp.float32),
            jax.ShapeDtypeStruct((1, E), jnp.int32),
            jax.ShapeDtypeStruct((S, 1), jnp.int32),
            jax.ShapeDtypeStruct((S, 1), jnp.int32),
        ),
    )(x, router_w, rb)

    G = (2 * S) // _BM + E
    c = counts.reshape(E)
    nb = (c + (_BM - 1)) // _BM
    nb_end = jnp.cumsum(nb)
    nblk = nb_end[-1]
    gids = jnp.arange(G, dtype=jnp.int32)
    be = jnp.searchsorted(nb_end, gids, side="right").astype(jnp.int32)
    bstart = nb_end - nb
    safe_e = jnp.minimum(be, E - 1)
    bj = gids - bstart[safe_e]
    last_e = jnp.clip(be[jnp.maximum(nblk - 1, 0)], 0, E - 1)
    be = jnp.where(gids < nblk, safe_e, last_e).astype(jnp.int32)
    bj = jnp.where(gids < nblk, bj, 0).astype(jnp.int32)
    nblk_arr = jnp.full((1,), nblk, dtype=jnp.int32)

    xs = _make_sc_scatter(S, D, G * _BM)(x, r1.reshape(S), r2.reshape(S))

    hchunk = H // _HC
    grid_spec = pltpu.PrefetchScalarGridSpec(
        num_scalar_prefetch=3,
        grid=(_HC, G),
        in_specs=[
            pl.BlockSpec((_BM, D), lambda hc, g, be, bj, nblk: (g, 0)),
            pl.BlockSpec((S, E), lambda hc, g, be, bj, nblk: (0, 0)),
            pl.BlockSpec((S, E), lambda hc, g, be, bj, nblk: (0, 0)),
            pl.BlockSpec((S, E), lambda hc, g, be, bj, nblk: (0, 0)),
            pl.BlockSpec((1, D, hchunk),
                         lambda hc, g, be, bj, nblk: (be[g], 0, hc)),
            pl.BlockSpec((1, 1, hchunk),
                         lambda hc, g, be, bj, nblk: (be[g], 0, hc)),
            pl.BlockSpec((1, hchunk, D),
                         lambda hc, g, be, bj, nblk: (be[g], hc, 0)),
            pl.BlockSpec((1, 1, D),
                         lambda hc, g, be, bj, nblk: (be[g], 0, 0)),
        ],
        out_specs=pl.BlockSpec((S, D),
                               lambda hc, g, be, bj, nblk: (0, 0)),
        scratch_shapes=[
            pltpu.VMEM((D, hchunk), jnp.bfloat16),
            pltpu.VMEM((hchunk, D), jnp.bfloat16),
        ],
    )
    out = pl.pallas_call(
        _ffn_body,
        grid_spec=grid_spec,
        out_shape=jax.ShapeDtypeStruct((S, D), jnp.float32),
        compiler_params=pltpu.CompilerParams(
            dimension_semantics=("arbitrary", "arbitrary")),
    )(
        be, bj, nblk_arr,
        xs,
        mask, pos, gates,
        fc1_w,
        fc1_b.reshape(E, 1, H),
        fc2_w,
        fc2_b.reshape(E, 1, D),
    )
    return out.reshape(B, S0, D)
